# Initial kernel scaffold; baseline (speedup 1.0000x reference)
import jax
import jax.numpy as jnp
from jax.experimental import pallas as pl


def kernel(x, edge_index, batch, params):
    raise NotImplementedError("write your pallas kernel here")


# speedup vs baseline: 4.0766x; 4.0766x over previous
import functools

import jax
import jax.numpy as jnp
from jax import lax
from jax.experimental import pallas as pl
from jax.experimental.pallas import tpu as pltpu
from jax.experimental.pallas import tpu_sc as plsc

N = 10000
E = 320000
D_IN = 128
H = 128
MLP_H = 64
G = 32

NC, NS = 2, 16
NW = NC * NS
EPW = E // NW
K = 80
NCHUNK = EPW // K
NPAD = 10240
ROWS_PT = NPAD // NS
CW = 128

BN = 400
GRID_N = N // BN

@functools.cache
def _mesh():
    return plsc.VectorSubcoreMesh(core_axis_name="c", subcore_axis_name="s",
                                  num_cores=NC, num_subcores=NS)


def _sc_agg_body(h_hbm, src_hbm, dst_hbm, out_hbm,
                 idx_v, dst_v, rows_v, stage_v, accum_sh, sem):
    c = lax.axis_index("c")
    s = lax.axis_index("s")
    wid = c * NS + s

    zero16 = jnp.zeros((16,), jnp.float32)
    for r in range(32):
        for j in range(H // 16):
            stage_v[r, pl.ds(j * 16, 16)] = zero16
    tbase = s * ROWS_PT
    for i in range(ROWS_PT // 32):
        pltpu.sync_copy(stage_v, accum_sh.at[pl.ds(tbase + i * 32, 32)])
    plsc.subcore_barrier()

    ebase = wid * EPW

    def chunk(i, _):
        off = ebase + i * K
        pltpu.sync_copy(src_hbm.at[pl.ds(off, K)], idx_v)
        pltpu.sync_copy(dst_hbm.at[pl.ds(off, K)], dst_v)
        pltpu.async_copy(h_hbm.at[idx_v], rows_v, sem).wait()
        pltpu.sync_copy(rows_v, accum_sh.at[dst_v], add=True)
        return 0

    lax.fori_loop(0, NCHUNK, chunk, 0)
    plsc.subcore_barrier()

    for i in range(ROWS_PT // 32):
        pltpu.sync_copy(accum_sh.at[pl.ds(tbase + i * 32, 32)], stage_v)
        pltpu.sync_copy(stage_v, out_hbm.at[c, pl.ds(tbase + i * 32, 32)])


@functools.cache
def _sc_agg_kernel():
    return pl.kernel(
        _sc_agg_body,
        out_type=jax.ShapeDtypeStruct((NC, NPAD, H), jnp.float32),
        mesh=_mesh(),
        scratch_types=[
            pltpu.VMEM((K,), jnp.int32),
            pltpu.VMEM((K,), jnp.int32),
            pltpu.VMEM((K, H), jnp.float32),
            pltpu.VMEM((32, H), jnp.float32),
            pltpu.VMEM_SHARED((NPAD, H), jnp.float32),
            pltpu.SemaphoreType.DMA,
        ],
    )


def _sc_agg(h, src, dst):
    return _sc_agg_kernel()(h, src, dst)


def _sc_cnt_body(dst_hbm, out_hbm, dst_v, ones_v, stage_v, accum_sh):
    c = lax.axis_index("c")
    s = lax.axis_index("s")
    wid = c * NS + s

    one16 = jnp.ones((16,), jnp.float32)
    zero16 = jnp.zeros((16,), jnp.float32)
    for r in range(K):
        for j in range(CW // 16):
            ones_v[r, pl.ds(j * 16, 16)] = one16
    for r in range(32):
        for j in range(CW // 16):
            stage_v[r, pl.ds(j * 16, 16)] = zero16
    tbase = s * ROWS_PT
    for i in range(ROWS_PT // 32):
        pltpu.sync_copy(stage_v, accum_sh.at[pl.ds(tbase + i * 32, 32)])
    plsc.subcore_barrier()

    ebase = wid * EPW

    def chunk(i, _):
        off = ebase + i * K
        pltpu.sync_copy(dst_hbm.at[pl.ds(off, K)], dst_v)
        pltpu.sync_copy(ones_v, accum_sh.at[dst_v], add=True)
        return 0

    lax.fori_loop(0, NCHUNK, chunk, 0)
    plsc.subcore_barrier()

    for i in range(ROWS_PT // 32):
        pltpu.sync_copy(accum_sh.at[pl.ds(tbase + i * 32, 32)], stage_v)
        pltpu.sync_copy(stage_v, out_hbm.at[c, pl.ds(tbase + i * 32, 32)])


@functools.cache
def _sc_cnt_kernel():
    return pl.kernel(
        _sc_cnt_body,
        out_type=jax.ShapeDtypeStruct((NC, NPAD, CW), jnp.float32),
        mesh=_mesh(),
        scratch_types=[
            pltpu.VMEM((K,), jnp.int32),
            pltpu.VMEM((K, CW), jnp.float32),
            pltpu.VMEM((32, CW), jnp.float32),
            pltpu.VMEM_SHARED((NPAD, CW), jnp.float32),
        ],
    )


def _sc_cnt(dst):
    return _sc_cnt_kernel()(dst)


def _ln(z, g, b):
    mu = jnp.mean(z, axis=-1, keepdims=True)
    var = jnp.mean(jnp.square(z - mu), axis=-1, keepdims=True)
    return (z - mu) * lax.rsqrt(var + 1e-5) * g + b


def _dot(a, b):
    return jnp.dot(a, b, preferred_element_type=jnp.float32)


def _enc_body(x_ref, w0, b0, w1, b1, w2, b2, lng, lnb, w3, b3, o_ref):
    h = jnp.maximum(_dot(x_ref[...], w0[...]) + b0[...], 0.0)
    h = jnp.maximum(_dot(h, w1[...]) + b1[...], 0.0)
    h = jnp.maximum(_dot(h, w2[...]) + b2[...], 0.0)
    h = _ln(h, lng[...], lnb[...])
    o_ref[...] = _dot(h, w3[...]) + b3[...]


def _full(shape):
    return pl.BlockSpec(shape, lambda i: (0,) * len(shape))


def _encoder(x, p):
    specs = [pl.BlockSpec((BN, D_IN), lambda i: (i, 0)),
             _full((D_IN, MLP_H)), _full((1, MLP_H)),
             _full((MLP_H, MLP_H)), _full((1, MLP_H)),
             _full((MLP_H, MLP_H)), _full((1, MLP_H)),
             _full((1, MLP_H)), _full((1, MLP_H)),
             _full((MLP_H, H)), _full((1, H))]
    return pl.pallas_call(
        _enc_body,
        grid=(GRID_N,),
        in_specs=specs,
        out_specs=pl.BlockSpec((BN, H), lambda i: (i, 0)),
        out_shape=jax.ShapeDtypeStruct((N, H), jnp.float32),
    )(x, p['enc_w0'], p['enc_b0'].reshape(1, -1),
      p['enc_w1'], p['enc_b1'].reshape(1, -1),
      p['enc_w2'], p['enc_b2'].reshape(1, -1),
      p['enc_ln_g'].reshape(1, -1), p['enc_ln_b'].reshape(1, -1),
      p['enc_w3'], p['enc_b3'].reshape(1, -1))


def _conv_body(p0, p1, c0, c1, h_ref, wl, bl, wr, o_ref):
    cnt = c0[:, :1] + c1[:, :1]
    inv = 1.0 / jnp.maximum(cnt, 1.0)
    mean = (p0[...] + p1[...]) * inv
    o_ref[...] = jnp.maximum(
        _dot(mean, wl[...]) + bl[...] + _dot(h_ref[...], wr[...]), 0.0)


def _conv(parts, cnt, h, wl, bl, wr):
    specs = [pl.BlockSpec((BN, H), lambda i: (i, 0)),
             pl.BlockSpec((BN, H), lambda i: (i, 0)),
             pl.BlockSpec((BN, CW), lambda i: (i, 0)),
             pl.BlockSpec((BN, CW), lambda i: (i, 0)),
             pl.BlockSpec((BN, H), lambda i: (i, 0)),
             _full((H, H)), _full((1, H)), _full((H, H))]
    return pl.pallas_call(
        _conv_body,
        grid=(GRID_N,),
        in_specs=specs,
        out_specs=pl.BlockSpec((BN, H), lambda i: (i, 0)),
        out_shape=jax.ShapeDtypeStruct((N, H), jnp.float32),
    )(parts[0], parts[1], cnt[0], cnt[1], h, wl, bl.reshape(1, -1), wr)


def _pool_dec_body(b_ref, h_ref,
                   ln0g0, ln0b0, w00, b00, ln1g0, ln1b0, w10, b10,
                   ln0g1, ln0b1, w01, b01, ln1g1, ln1b1, w11, b11,
                   o_ref, acc_ref):
    i = pl.program_id(0)
    bv = jnp.broadcast_to(b_ref[0], (G, BN))
    ids = lax.broadcasted_iota(jnp.int32, (G, BN), 0)
    oh = jnp.where(ids == bv, 1.0, 0.0)
    part = lax.dot_general(oh, h_ref[...], (((1,), (0,)), ((), ())),
                           preferred_element_type=jnp.float32)

    @pl.when(i == 0)
    def _():
        acc_ref[...] = part

    @pl.when(i > 0)
    def _():
        acc_ref[...] = acc_ref[...] + part

    @pl.when(i == GRID_N - 1)
    def _():
        pooled = acc_ref[...]
        z0 = _ln(pooled, ln0g0[...], ln0b0[...])
        z0 = jnp.maximum(_dot(z0, w00[...]) + b00[...], 0.0)
        z0 = _ln(z0, ln1g0[...], ln1b0[...])
        z0 = jnp.maximum(_dot(z0, w10[...]) + b10[...], 0.0)
        z1 = _ln(pooled, ln0g1[...], ln0b1[...])
        z1 = jnp.maximum(_dot(z1, w01[...]) + b01[...], 0.0)
        z1 = _ln(z1, ln1g1[...], ln1b1[...])
        z1 = jnp.maximum(_dot(z1, w11[...]) + b11[...], 0.0)
        o_ref[...] = z0 + z1


def _pool_decode(batch3d, h, p):
    ins = [batch3d, h]
    for hd in range(2):
        w1 = p['dec%d_w1' % hd]
        b1 = p['dec%d_b1' % hd]
        sel = (jnp.arange(H, dtype=jnp.float32) == hd).reshape(1, H)
        ins += [p['dec%d_ln0_g' % hd].reshape(1, -1),
                p['dec%d_ln0_b' % hd].reshape(1, -1),
                p['dec%d_w0' % hd], p['dec%d_b0' % hd].reshape(1, -1),
                p['dec%d_ln1_g' % hd].reshape(1, -1),
                p['dec%d_ln1_b' % hd].reshape(1, -1),
                w1 @ sel, b1.reshape(1, 1) @ sel]
    specs = [pl.BlockSpec((1, 1, BN), lambda i: (i, 0, 0)),
             pl.BlockSpec((BN, H), lambda i: (i, 0))]
    for hd in range(2):
        specs += [_full((1, H)), _full((1, H)), _full((H, H)), _full((1, H)),
                  _full((1, H)), _full((1, H)), _full((H, H)), _full((1, H))]
    out = pl.pallas_call(
        _pool_dec_body,
        grid=(GRID_N,),
        in_specs=specs,
        out_specs=pl.BlockSpec((G, H), lambda i: (0, 0)),
        out_shape=jax.ShapeDtypeStruct((G, H), jnp.float32),
        scratch_shapes=[pltpu.VMEM((G, H), jnp.float32)],
    )(*ins)
    return out[:, :2]


def kernel(x, edge_index, batch, params):
    src = edge_index[0]
    dst = edge_index[1]
    h = _encoder(x, params)
    cnt = _sc_cnt(dst)
    for i in range(3):
        parts = _sc_agg(h, src, dst)
        h = _conv(parts, cnt, h,
                  params['conv%d_wl' % i], params['conv%d_bl' % i],
                  params['conv%d_wr' % i])
    return _pool_decode(batch.reshape(GRID_N, 1, BN), h, params)


# speedup vs baseline: 6.5959x; 1.6180x over previous
import functools

import jax
import jax.numpy as jnp
from jax import lax
from jax.experimental import pallas as pl
from jax.experimental.pallas import tpu as pltpu
from jax.experimental.pallas import tpu_sc as plsc

N = 10000
E = 320000
D_IN = 128
H = 128
MLP_H = 64
G = 32

NC, NS = 2, 16
NW = NC * NS
EPW = E // NW
K = 80
NCHUNK = EPW // K
NPAD = 10240
ROWS_PT = NPAD // NS
CW = 128

BN = 400
GRID_N = N // BN

@functools.cache
def _mesh():
    return plsc.VectorSubcoreMesh(core_axis_name="c", subcore_axis_name="s",
                                  num_cores=NC, num_subcores=NS)


def _sc_agg_body(h_hbm, edges_hbm, out_hbm,
                 idx_a, idx_b, rows_a, rows_b, stage_v, accum_sh,
                 sem_ia, sem_ib, sem_a, sem_b):
    c = lax.axis_index("c")
    s = lax.axis_index("s")
    wid = c * NS + s

    def issue_idx(i, idx, sem):
        pltpu.async_copy(edges_hbm.at[wid, i], idx, sem)

    def issue_gather(i, idx, sem_i, rows, sem_r):
        pltpu.make_async_copy(edges_hbm.at[wid, i], idx, sem_i).wait()
        pltpu.async_copy(h_hbm.at[idx.at[0]], rows, sem_r)

    def do_scatter(i, idx, rows, sem_r):
        pltpu.make_async_copy(h_hbm.at[idx.at[0]], rows, sem_r).wait()
        pltpu.sync_copy(rows, accum_sh.at[idx.at[1]], add=True)

    issue_idx(0, idx_a, sem_ia)
    issue_idx(1, idx_b, sem_ib)

    zero16 = jnp.zeros((16,), jnp.float32)
    for r in range(32):
        for j in range(H // 16):
            stage_v[r, pl.ds(j * 16, 16)] = zero16
    tbase = s * ROWS_PT
    for i in range(ROWS_PT // 32):
        pltpu.sync_copy(stage_v, accum_sh.at[pl.ds(tbase + i * 32, 32)])
    plsc.subcore_barrier()

    issue_gather(0, idx_a, sem_ia, rows_a, sem_a)
    issue_gather(1, idx_b, sem_ib, rows_b, sem_b)

    def pair(j, _):
        ia = 2 * j
        ib = ia + 1
        do_scatter(ia, idx_a, rows_a, sem_a)

        @pl.when(ia + 2 < NCHUNK)
        def _():
            issue_idx(ia + 2, idx_a, sem_ia)
            issue_gather(ia + 2, idx_a, sem_ia, rows_a, sem_a)

        do_scatter(ib, idx_b, rows_b, sem_b)

        @pl.when(ib + 2 < NCHUNK)
        def _():
            issue_idx(ib + 2, idx_b, sem_ib)
            issue_gather(ib + 2, idx_b, sem_ib, rows_b, sem_b)

        return 0

    lax.fori_loop(0, NCHUNK // 2, pair, 0)
    if NCHUNK % 2:
        do_scatter(NCHUNK - 1, idx_a, rows_a, sem_a)
    plsc.subcore_barrier()

    for i in range(ROWS_PT // 32):
        pltpu.sync_copy(accum_sh.at[pl.ds(tbase + i * 32, 32)], stage_v)
        pltpu.sync_copy(stage_v, out_hbm.at[c, pl.ds(tbase + i * 32, 32)])


@functools.cache
def _sc_agg_kernel():
    return pl.kernel(
        _sc_agg_body,
        out_type=jax.ShapeDtypeStruct((NC, NPAD, H), jnp.float32),
        mesh=_mesh(),
        scratch_types=[
            pltpu.VMEM((2, K), jnp.int32),
            pltpu.VMEM((2, K), jnp.int32),
            pltpu.VMEM((K, H), jnp.float32),
            pltpu.VMEM((K, H), jnp.float32),
            pltpu.VMEM((32, H), jnp.float32),
            pltpu.VMEM_SHARED((NPAD, H), jnp.float32),
            pltpu.SemaphoreType.DMA,
            pltpu.SemaphoreType.DMA,
            pltpu.SemaphoreType.DMA,
            pltpu.SemaphoreType.DMA,
        ],
    )


def _sc_agg(h, src, dst):
    edges = jnp.stack([src.reshape(NW, NCHUNK, K),
                       dst.reshape(NW, NCHUNK, K)], axis=2)
    return _sc_agg_kernel()(h, edges)


def _sc_cnt_body(dst_hbm, out_hbm, dst_v, ones_v, stage_v, accum_sh):
    c = lax.axis_index("c")
    s = lax.axis_index("s")
    wid = c * NS + s

    one16 = jnp.ones((16,), jnp.float32)
    zero16 = jnp.zeros((16,), jnp.float32)
    for r in range(K):
        for j in range(CW // 16):
            ones_v[r, pl.ds(j * 16, 16)] = one16
    for r in range(32):
        for j in range(CW // 16):
            stage_v[r, pl.ds(j * 16, 16)] = zero16
    tbase = s * ROWS_PT
    for i in range(ROWS_PT // 32):
        pltpu.sync_copy(stage_v, accum_sh.at[pl.ds(tbase + i * 32, 32)])
    plsc.subcore_barrier()

    ebase = wid * EPW

    def chunk(i, _):
        off = ebase + i * K
        pltpu.sync_copy(dst_hbm.at[pl.ds(off, K)], dst_v)
        pltpu.sync_copy(ones_v, accum_sh.at[dst_v], add=True)
        return 0

    lax.fori_loop(0, NCHUNK, chunk, 0)
    plsc.subcore_barrier()

    for i in range(ROWS_PT // 32):
        pltpu.sync_copy(accum_sh.at[pl.ds(tbase + i * 32, 32)], stage_v)
        pltpu.sync_copy(stage_v, out_hbm.at[c, pl.ds(tbase + i * 32, 32)])


@functools.cache
def _sc_cnt_kernel():
    return pl.kernel(
        _sc_cnt_body,
        out_type=jax.ShapeDtypeStruct((NC, NPAD, CW), jnp.float32),
        mesh=_mesh(),
        scratch_types=[
            pltpu.VMEM((K,), jnp.int32),
            pltpu.VMEM((K, CW), jnp.float32),
            pltpu.VMEM((32, CW), jnp.float32),
            pltpu.VMEM_SHARED((NPAD, CW), jnp.float32),
        ],
    )


def _sc_cnt(dst):
    return _sc_cnt_kernel()(dst)


def _ln(z, g, b):
    mu = jnp.mean(z, axis=-1, keepdims=True)
    var = jnp.mean(jnp.square(z - mu), axis=-1, keepdims=True)
    return (z - mu) * lax.rsqrt(var + 1e-5) * g + b


def _dot(a, b):
    return jnp.dot(a, b, preferred_element_type=jnp.float32)


def _enc_body(x_ref, w0, b0, w1, b1, w2, b2, lng, lnb, w3, b3, o_ref):
    h = jnp.maximum(_dot(x_ref[...], w0[...]) + b0[...], 0.0)
    h = jnp.maximum(_dot(h, w1[...]) + b1[...], 0.0)
    h = jnp.maximum(_dot(h, w2[...]) + b2[...], 0.0)
    h = _ln(h, lng[...], lnb[...])
    o_ref[...] = _dot(h, w3[...]) + b3[...]


def _full(shape):
    return pl.BlockSpec(shape, lambda i: (0,) * len(shape))


def _encoder(x, p):
    specs = [pl.BlockSpec((BN, D_IN), lambda i: (i, 0)),
             _full((D_IN, MLP_H)), _full((1, MLP_H)),
             _full((MLP_H, MLP_H)), _full((1, MLP_H)),
             _full((MLP_H, MLP_H)), _full((1, MLP_H)),
             _full((1, MLP_H)), _full((1, MLP_H)),
             _full((MLP_H, H)), _full((1, H))]
    return pl.pallas_call(
        _enc_body,
        grid=(GRID_N,),
        in_specs=specs,
        out_specs=pl.BlockSpec((BN, H), lambda i: (i, 0)),
        out_shape=jax.ShapeDtypeStruct((N, H), jnp.float32),
    )(x, p['enc_w0'], p['enc_b0'].reshape(1, -1),
      p['enc_w1'], p['enc_b1'].reshape(1, -1),
      p['enc_w2'], p['enc_b2'].reshape(1, -1),
      p['enc_ln_g'].reshape(1, -1), p['enc_ln_b'].reshape(1, -1),
      p['enc_w3'], p['enc_b3'].reshape(1, -1))


def _conv_body(p0, p1, c0, c1, h_ref, wl, bl, wr, o_ref):
    cnt = c0[:, :1] + c1[:, :1]
    inv = 1.0 / jnp.maximum(cnt, 1.0)
    mean = (p0[...] + p1[...]) * inv
    o_ref[...] = jnp.maximum(
        _dot(mean, wl[...]) + bl[...] + _dot(h_ref[...], wr[...]), 0.0)


def _conv(parts, cnt, h, wl, bl, wr):
    specs = [pl.BlockSpec((BN, H), lambda i: (i, 0)),
             pl.BlockSpec((BN, H), lambda i: (i, 0)),
             pl.BlockSpec((BN, CW), lambda i: (i, 0)),
             pl.BlockSpec((BN, CW), lambda i: (i, 0)),
             pl.BlockSpec((BN, H), lambda i: (i, 0)),
             _full((H, H)), _full((1, H)), _full((H, H))]
    return pl.pallas_call(
        _conv_body,
        grid=(GRID_N,),
        in_specs=specs,
        out_specs=pl.BlockSpec((BN, H), lambda i: (i, 0)),
        out_shape=jax.ShapeDtypeStruct((N, H), jnp.float32),
    )(parts[0], parts[1], cnt[0], cnt[1], h, wl, bl.reshape(1, -1), wr)


def _pool_dec_body(b_ref, h_ref,
                   ln0g0, ln0b0, w00, b00, ln1g0, ln1b0, w10, b10,
                   ln0g1, ln0b1, w01, b01, ln1g1, ln1b1, w11, b11,
                   o_ref, acc_ref):
    i = pl.program_id(0)
    bv = jnp.broadcast_to(b_ref[0], (G, BN))
    ids = lax.broadcasted_iota(jnp.int32, (G, BN), 0)
    oh = jnp.where(ids == bv, 1.0, 0.0)
    part = lax.dot_general(oh, h_ref[...], (((1,), (0,)), ((), ())),
                           preferred_element_type=jnp.float32)

    @pl.when(i == 0)
    def _():
        acc_ref[...] = part

    @pl.when(i > 0)
    def _():
        acc_ref[...] = acc_ref[...] + part

    @pl.when(i == GRID_N - 1)
    def _():
        pooled = acc_ref[...]
        z0 = _ln(pooled, ln0g0[...], ln0b0[...])
        z0 = jnp.maximum(_dot(z0, w00[...]) + b00[...], 0.0)
        z0 = _ln(z0, ln1g0[...], ln1b0[...])
        z0 = jnp.maximum(_dot(z0, w10[...]) + b10[...], 0.0)
        z1 = _ln(pooled, ln0g1[...], ln0b1[...])
        z1 = jnp.maximum(_dot(z1, w01[...]) + b01[...], 0.0)
        z1 = _ln(z1, ln1g1[...], ln1b1[...])
        z1 = jnp.maximum(_dot(z1, w11[...]) + b11[...], 0.0)
        o_ref[...] = z0 + z1


def _pool_decode(batch3d, h, p):
    ins = [batch3d, h]
    for hd in range(2):
        w1 = p['dec%d_w1' % hd]
        b1 = p['dec%d_b1' % hd]
        sel = (jnp.arange(H, dtype=jnp.float32) == hd).reshape(1, H)
        ins += [p['dec%d_ln0_g' % hd].reshape(1, -1),
                p['dec%d_ln0_b' % hd].reshape(1, -1),
                p['dec%d_w0' % hd], p['dec%d_b0' % hd].reshape(1, -1),
                p['dec%d_ln1_g' % hd].reshape(1, -1),
                p['dec%d_ln1_b' % hd].reshape(1, -1),
                w1 @ sel, b1.reshape(1, 1) @ sel]
    specs = [pl.BlockSpec((1, 1, BN), lambda i: (i, 0, 0)),
             pl.BlockSpec((BN, H), lambda i: (i, 0))]
    for hd in range(2):
        specs += [_full((1, H)), _full((1, H)), _full((H, H)), _full((1, H)),
                  _full((1, H)), _full((1, H)), _full((H, H)), _full((1, H))]
    out = pl.pallas_call(
        _pool_dec_body,
        grid=(GRID_N,),
        in_specs=specs,
        out_specs=pl.BlockSpec((G, H), lambda i: (0, 0)),
        out_shape=jax.ShapeDtypeStruct((G, H), jnp.float32),
        scratch_shapes=[pltpu.VMEM((G, H), jnp.float32)],
    )(*ins)
    return out[:, :2]


def kernel(x, edge_index, batch, params):
    src = edge_index[0]
    dst = edge_index[1]
    h = _encoder(x, params)
    cnt = _sc_cnt(dst)
    for i in range(3):
        parts = _sc_agg(h, src, dst)
        h = _conv(parts, cnt, h,
                  params['conv%d_wl' % i], params['conv%d_bl' % i],
                  params['conv%d_wr' % i])
    return _pool_decode(batch.reshape(GRID_N, 1, BN), h, params)
